# baseline (speedup 1.0000x reference)
import functools

import jax
import jax.numpy as jnp
from jax import lax
from jax.experimental import pallas as pl
from jax.experimental.pallas import tpu as pltpu
from jax.experimental.pallas import tpu_sc as plsc

SUB = 16
LANES = 16
MAGIC = jnp.int32(0x5F3759DF)


def _newton_rsqrt(x):
    i = plsc.bitcast(x, jnp.int32)
    y = plsc.bitcast(MAGIC - lax.shift_right_logical(i, 1), jnp.float32)
    for _ in range(3):
        y = y * (1.5 - 0.5 * x * y * y)
    return y


def _make_sc_propagate(n, e):
    npad = ((n + SUB * LANES - 1) // (SUB * LANES)) * (SUB * LANES)
    ns = npad // SUB
    ec = e // SUB
    assert ec * SUB == e and ec % LANES == 0
    ve = ec // LANES
    nvec = npad // LANES
    nsv = ns // LANES

    mesh = plsc.VectorSubcoreMesh(
        core_axis_name="c", subcore_axis_name="s", num_cores=1)

    @functools.partial(
        pl.kernel,
        out_type=jax.ShapeDtypeStruct((4, npad), jnp.float32),
        mesh=mesh,
        compiler_params=pltpu.CompilerParams(needs_layout_passes=False),
        scratch_types=[
            pltpu.VMEM((ec,), jnp.int32),
            pltpu.VMEM((ec,), jnp.int32),
            pltpu.VMEM((npad,), jnp.float32),
            pltpu.VMEM((npad,), jnp.float32),
            pltpu.VMEM((npad,), jnp.float32),
            pltpu.VMEM((SUB, ns), jnp.float32),
            pltpu.VMEM((ns,), jnp.float32),
            pltpu.VMEM((ns,), jnp.float32),
            pltpu.VMEM((ns,), jnp.float32),
            pltpu.VMEM_SHARED((SUB, npad), jnp.float32),
            pltpu.VMEM_SHARED((npad,), jnp.float32),
            pltpu.VMEM_SHARED((npad,), jnp.float32),
            pltpu.SemaphoreType.DMA,
            pltpu.SemaphoreType.DMA,
        ],
    )
    def sc_propagate(src_hbm, dst_hbm, t_hbm, srcv, dstv, qv, dinvv,
                     accv, blkv, redv, tstg, qstg, sh_acc, sh_t, sh_q,
                     sem, sem2):
        wid = lax.axis_index("s")
        eb = wid * ec
        nb = wid * ns
        zeros = jnp.zeros((LANES,), jnp.float32)
        ones = jnp.full((LANES,), 1.0, jnp.float32)

        pltpu.sync_copy(src_hbm.at[pl.ds(eb, ec)], srcv)
        pltpu.sync_copy(dst_hbm.at[pl.ds(eb, ec)], dstv)

        def zero_acc():
            @plsc.parallel_loop(0, nvec, unroll=8)
            def _(i):
                accv[pl.ds(i * LANES, LANES)] = zeros

        def publish_reduce():
            pltpu.sync_copy(accv, sh_acc.at[wid])
            plsc.subcore_barrier()
            plsc.subcore_barrier()
            for t in range(SUB):
                pltpu.make_async_copy(
                    sh_acc.at[t, pl.ds(nb, ns)], blkv.at[t], sem).start()
            for t in range(SUB):
                pltpu.make_async_copy(
                    sh_acc.at[t, pl.ds(nb, ns)], blkv.at[t], sem).wait()

            @plsc.parallel_loop(0, nsv, unroll=2)
            def _(j):
                b = j * LANES
                s = blkv[0, pl.ds(b, LANES)]
                for t in range(1, SUB):
                    s = s + blkv[t, pl.ds(b, LANES)]
                redv[pl.ds(b, LANES)] = s

        zero_acc()

        @plsc.parallel_loop(0, ve, unroll=16)
        def _(i):
            b = i * LANES
            plsc.addupdate_scatter(accv, [srcv[pl.ds(b, LANES)]], ones)
            plsc.addupdate_scatter(accv, [dstv[pl.ds(b, LANES)]], ones)

        publish_reduce()
        for j in range(nsv):
            b = j * LANES
            d = jnp.maximum(redv[pl.ds(b, LANES)], 1.0)
            tstg[pl.ds(b, LANES)] = _newton_rsqrt(d)
        pltpu.sync_copy(tstg, sh_q.at[pl.ds(nb, ns)])
        plsc.subcore_barrier()
        plsc.subcore_barrier()
        pltpu.make_async_copy(sh_q, dinvv, sem2).start()
        pltpu.make_async_copy(sh_q, qv, sem2).start()

        for r in range(4):
            zero_acc()
            if r == 0:
                pltpu.make_async_copy(sh_q, dinvv, sem2).wait()
            pltpu.make_async_copy(sh_q, qv, sem2).wait()

            @plsc.parallel_loop(0, ve, unroll=24)
            def _(i):
                b = i * LANES
                vals = plsc.load_gather(qv, [dstv[pl.ds(b, LANES)]])
                plsc.addupdate_scatter(accv, [srcv[pl.ds(b, LANES)]], vals)

            publish_reduce()
            last = r == 3
            for j in range(nsv):
                b = j * LANES
                a = redv[pl.ds(b, LANES)]
                dv = dinvv[pl.ds(nb + b, LANES)]
                t = a * dv
                tstg[pl.ds(b, LANES)] = t
                if not last:
                    qstg[pl.ds(b, LANES)] = t * dv
            pltpu.sync_copy(tstg, sh_t.at[pl.ds(nb, ns)])
            if not last:
                pltpu.sync_copy(qstg, sh_q.at[pl.ds(nb, ns)])
            plsc.subcore_barrier()
            plsc.subcore_barrier()

            @pl.when(wid == 0)
            def _():
                pltpu.sync_copy(sh_t, t_hbm.at[r])

            if not last:
                pltpu.make_async_copy(sh_q, qv, sem2).start()

    return sc_propagate


def _tc_head(t, x, w1r, b1r, w2, b2r):
    n = x.shape[0]
    inv_n = 1.0 / n

    def body(t_ref, x_ref, w1_ref, b1_ref, w2_ref, b2_ref, o_ref):
        dn = (((1,), (0,)), ((), ()))
        tt = t_ref[...][:, :n]
        g = lax.dot_general(tt, x_ref[...], dn,
                            preferred_element_type=jnp.float32) * inv_n
        acc = b1_ref[...]
        for k in range(4):
            acc = acc + lax.dot_general(g[k:k + 1, :], w1_ref[k], dn,
                                        preferred_element_type=jnp.float32)
        z = jnp.maximum(acc, 0.0)
        y = lax.dot_general(z, w2_ref[...], dn,
                            preferred_element_type=jnp.float32) + b2_ref[...]
        o_ref[...] = 1.0 / (1.0 + jnp.exp(-y))

    return pl.pallas_call(
        body,
        out_shape=jax.ShapeDtypeStruct((1, 1), jnp.float32),
    )(t, x, w1r, b1r, w2, b2r)


def kernel(x, edge_index, W1, b1, W2, b2):
    n = x.shape[0]
    e = edge_index.shape[1]
    t = _make_sc_propagate(n, e)(edge_index[0], edge_index[1])
    out = _tc_head(t, x,
                   W1.reshape(4, x.shape[1], W1.shape[1]),
                   b1.reshape(1, -1), W2, b2.reshape(1, 1))
    return out[0, 0]


# speedup vs baseline: 1.0291x; 1.0291x over previous
import functools

import jax
import jax.numpy as jnp
from jax import lax
from jax.experimental import pallas as pl
from jax.experimental.pallas import tpu as pltpu
from jax.experimental.pallas import tpu_sc as plsc

SUB = 16
LANES = 16
MAGIC = jnp.int32(0x5F3759DF)


def _newton_rsqrt(x):
    i = plsc.bitcast(x, jnp.int32)
    y = plsc.bitcast(MAGIC - lax.shift_right_logical(i, 1), jnp.float32)
    for _ in range(3):
        y = y * (1.5 - 0.5 * x * y * y)
    return y


def _make_sc_propagate(n, e):
    npad = ((n + SUB * LANES - 1) // (SUB * LANES)) * (SUB * LANES)
    ns = npad // SUB
    ec = e // SUB
    assert ec * SUB == e and ec % LANES == 0
    ve = ec // LANES
    nvec = npad // LANES
    nsv = ns // LANES

    mesh = plsc.VectorSubcoreMesh(
        core_axis_name="c", subcore_axis_name="s", num_cores=1)

    @functools.partial(
        pl.kernel,
        out_type=jax.ShapeDtypeStruct((4, npad), jnp.float32),
        mesh=mesh,
        compiler_params=pltpu.CompilerParams(needs_layout_passes=False),
        scratch_types=[
            pltpu.VMEM((ec,), jnp.int32),
            pltpu.VMEM((ec,), jnp.int32),
            pltpu.VMEM((npad,), jnp.float32),
            pltpu.VMEM((npad,), jnp.float32),
            pltpu.VMEM((npad,), jnp.float32),
            pltpu.VMEM((SUB, ns), jnp.float32),
            pltpu.VMEM((ns,), jnp.float32),
            pltpu.VMEM((ns,), jnp.float32),
            pltpu.VMEM((ns,), jnp.float32),
            pltpu.VMEM_SHARED((SUB, npad), jnp.float32),
            pltpu.VMEM_SHARED((npad,), jnp.float32),
            pltpu.VMEM_SHARED((npad,), jnp.float32),
            pltpu.SemaphoreType.DMA,
            pltpu.SemaphoreType.DMA,
        ],
    )
    def sc_propagate(src_hbm, dst_hbm, t_hbm, srcv, dstv, qv, dinvv,
                     accv, blkv, redv, tstg, qstg, sh_acc, sh_t, sh_q,
                     sem, sem2):
        wid = lax.axis_index("s")
        eb = wid * ec
        nb = wid * ns
        zeros = jnp.zeros((LANES,), jnp.float32)
        ones = jnp.full((LANES,), 1.0, jnp.float32)

        pltpu.sync_copy(src_hbm.at[pl.ds(eb, ec)], srcv)
        pltpu.sync_copy(dst_hbm.at[pl.ds(eb, ec)], dstv)

        def zero_acc():
            @plsc.parallel_loop(0, nvec, unroll=8)
            def _(i):
                accv[pl.ds(i * LANES, LANES)] = zeros

        def publish_reduce():
            pltpu.sync_copy(accv, sh_acc.at[wid])
            plsc.subcore_barrier()
            plsc.subcore_barrier()
            pltpu.sync_copy(sh_acc.at[:, pl.ds(nb, ns)], blkv)

            @plsc.parallel_loop(0, nsv, unroll=2)
            def _(j):
                b = j * LANES
                s = blkv[0, pl.ds(b, LANES)]
                for t in range(1, SUB):
                    s = s + blkv[t, pl.ds(b, LANES)]
                redv[pl.ds(b, LANES)] = s

        zero_acc()

        @plsc.parallel_loop(0, ve, unroll=8)
        def _(i):
            b = i * LANES
            plsc.addupdate_scatter(accv, [srcv[pl.ds(b, LANES)]], ones)
            plsc.addupdate_scatter(accv, [dstv[pl.ds(b, LANES)]], ones)

        publish_reduce()
        for j in range(nsv):
            b = j * LANES
            d = jnp.maximum(redv[pl.ds(b, LANES)], 1.0)
            tstg[pl.ds(b, LANES)] = _newton_rsqrt(d)
        pltpu.sync_copy(tstg, sh_q.at[pl.ds(nb, ns)])
        plsc.subcore_barrier()
        plsc.subcore_barrier()
        pltpu.make_async_copy(sh_q, dinvv, sem2).start()
        pltpu.make_async_copy(sh_q, qv, sem2).start()

        for r in range(4):
            zero_acc()
            if r == 0:
                pltpu.make_async_copy(sh_q, dinvv, sem2).wait()
            pltpu.make_async_copy(sh_q, qv, sem2).wait()

            @plsc.parallel_loop(0, ve, unroll=16)
            def _(i):
                b = i * LANES
                vals = plsc.load_gather(qv, [dstv[pl.ds(b, LANES)]])
                plsc.addupdate_scatter(accv, [srcv[pl.ds(b, LANES)]], vals)

            publish_reduce()
            last = r == 3
            for j in range(nsv):
                b = j * LANES
                a = redv[pl.ds(b, LANES)]
                dv = dinvv[pl.ds(nb + b, LANES)]
                t = a * dv
                tstg[pl.ds(b, LANES)] = t
                if not last:
                    qstg[pl.ds(b, LANES)] = t * dv
            pltpu.sync_copy(tstg, sh_t.at[pl.ds(nb, ns)])
            if not last:
                pltpu.sync_copy(qstg, sh_q.at[pl.ds(nb, ns)])
            plsc.subcore_barrier()
            plsc.subcore_barrier()

            @pl.when(wid == 0)
            def _():
                pltpu.sync_copy(sh_t, t_hbm.at[r])

            if not last:
                pltpu.make_async_copy(sh_q, qv, sem2).start()

    return sc_propagate


def _tc_head(t, x, w1r, b1r, w2, b2r):
    n = x.shape[0]
    inv_n = 1.0 / n

    def body(t_ref, x_ref, w1_ref, b1_ref, w2_ref, b2_ref, o_ref):
        dn = (((1,), (0,)), ((), ()))
        tt = t_ref[...][:, :n]
        g = lax.dot_general(tt, x_ref[...], dn,
                            preferred_element_type=jnp.float32) * inv_n
        acc = b1_ref[...]
        for k in range(4):
            acc = acc + lax.dot_general(g[k:k + 1, :], w1_ref[k], dn,
                                        preferred_element_type=jnp.float32)
        z = jnp.maximum(acc, 0.0)
        y = lax.dot_general(z, w2_ref[...], dn,
                            preferred_element_type=jnp.float32) + b2_ref[...]
        o_ref[...] = 1.0 / (1.0 + jnp.exp(-y))

    return pl.pallas_call(
        body,
        out_shape=jax.ShapeDtypeStruct((1, 1), jnp.float32),
    )(t, x, w1r, b1r, w2, b2r)


def kernel(x, edge_index, W1, b1, W2, b2):
    n = x.shape[0]
    e = edge_index.shape[1]
    t = _make_sc_propagate(n, e)(edge_index[0], edge_index[1])
    out = _tc_head(t, x,
                   W1.reshape(4, x.shape[1], W1.shape[1]),
                   b1.reshape(1, -1), W2, b2.reshape(1, 1))
    return out[0, 0]
